# baseline (speedup 1.0000x reference)
import functools

import jax
import jax.numpy as jnp
from jax import lax
from jax.experimental import pallas as pl
from jax.experimental.pallas import tpu as pltpu
from jax.experimental.pallas import tpu_sc as plsc

N_NODES = 10000
N_EDGES = 320000
D_FEAT = 128
D_EDGE = 16

NC = 2
NS = 16

N_EB = N_EDGES // 128
CH = 125
N_CH = N_EB // CH
N_PAD = 10240
NRB = N_PAD // 128


def _sc_body(e4_hbm, idx_hbm, out_hbm,
             idx_sh, val_v, idx_v, acc, stg, vs0, vs1, is0, is1):
    c = lax.axis_index("c")
    s = lax.axis_index("s")
    fh = s // 8
    fm = lax.rem(s, 8)

    @pl.when(s == 0)
    def _():
        pltpu.sync_copy(idx_hbm, idx_sh)

    def _zero(i, carry):
        acc[pl.ds(i * 16, 16)] = jnp.zeros((16,), jnp.float32)
        return carry
    lax.fori_loop(0, N_PAD // 16, _zero, 0)
    plsc.subcore_barrier()

    vsems = (vs0, vs1)
    isems = (is0, is1)

    def _fetch(ch, b):
        pltpu.async_copy(
            e4_hbm.at[fh, pl.ds(ch * CH, CH), pl.ds(fm, 1), :],
            val_v.at[b], vsems[b])
        pltpu.async_copy(idx_sh.at[pl.ds(ch * CH, CH), pl.ds(1 - c, 1), :],
                         idx_v.at[b], isems[b])

    def _process(b):
        pltpu.make_async_copy(
            e4_hbm.at[fh, pl.ds(0, CH), pl.ds(fm, 1), :],
            val_v.at[b], vsems[b]).wait()
        pltpu.make_async_copy(idx_sh.at[pl.ds(0, CH), pl.ds(0, 1), :],
                              idx_v.at[b], isems[b]).wait()

        def _scat(r):
            ivs = [idx_v[b, r, 0, pl.ds(u * 16, 16)] for u in range(8)]
            vvs = [val_v[b, r, 0, pl.ds(u * 16, 16)] for u in range(8)]
            for u in range(8):
                plsc.addupdate_scatter(acc, [ivs[u]], vvs[u])

        def _rowpair(rr, carry):
            _scat(2 * rr)
            _scat(2 * rr + 1)
            return carry
        lax.fori_loop(0, CH // 2, _rowpair, 0)
        _scat(CH - 1)

    _fetch(0, 0)
    _fetch(1, 1)

    def _pair(t, carry):
        _process(0)

        @pl.when(2 * t + 2 < N_CH)
        def _():
            _fetch(2 * t + 2, 0)
        _process(1)

        @pl.when(2 * t + 3 < N_CH)
        def _():
            _fetch(2 * t + 3, 1)
        return carry
    lax.fori_loop(0, N_CH // 2, _pair, 0)

    def _rp(r, carry):
        for u in range(8):
            stg[r, pl.ds(u * 16, 16)] = acc[pl.ds(r * 128 + u * 16, 16)]
        return carry
    lax.fori_loop(0, NRB, _rp, 0)
    pltpu.sync_copy(stg, out_hbm.at[c, s])


_sc_agg = pl.kernel(
    _sc_body,
    out_type=jax.ShapeDtypeStruct((NC, NS, NRB, 128), jnp.float32),
    mesh=plsc.VectorSubcoreMesh(core_axis_name="c", subcore_axis_name="s"),
    compiler_params=pltpu.CompilerParams(use_tc_tiling_on_sc=False,
                                         needs_layout_passes=False),
    scratch_types=[
        pltpu.VMEM_SHARED((N_EB, 2, 128), jnp.int32),
        pltpu.VMEM((2, CH, 1, 128), jnp.float32),
        pltpu.VMEM((2, CH, 1, 128), jnp.int32),
        pltpu.VMEM((N_PAD,), jnp.float32),
        pltpu.VMEM((NRB, 128), jnp.float32),
        pltpu.SemaphoreType.DMA,
        pltpu.SemaphoreType.DMA,
        pltpu.SemaphoreType.DMA,
        pltpu.SemaphoreType.DMA,
    ],
)


ROW_BLK = 1024


def _mlp_body(agg_ref, x_ref, wio_ref, wx_ref, b_ref, o_ref):
    base = jnp.dot(x_ref[...], wx_ref[...],
                   preferred_element_type=jnp.float32) + b_ref[...]
    ac = agg_ref[...].reshape(2 * D_EDGE, 8, 128)
    for j in range(8):
        aj = ac[:, j, :]
        part = lax.dot_general(aj, wio_ref[...], (((0,), (0,)), ((), ())),
                               preferred_element_type=jnp.float32)
        o_ref[pl.ds(j * 128, 128), :] = base[j * 128:(j + 1) * 128, :] + part


@functools.partial(jax.jit, static_argnames=())
def kernel(x, edge_attr, W, b, edge_index):
    e4 = edge_attr.T.reshape(NC, 8, N_EB, 128).transpose(0, 2, 1, 3)
    idx3 = edge_index.astype(jnp.int32).reshape(2, N_EB, 128).transpose(1, 0, 2)
    agg = _sc_agg(e4, idx3)

    wio = W[:2 * D_EDGE]
    wx = W[2 * D_EDGE:]
    b2 = b.reshape(1, D_FEAT)

    grid = (N_PAD // ROW_BLK,)
    out = pl.pallas_call(
        _mlp_body,
        grid=grid,
        in_specs=[
            pl.BlockSpec((NC, NS, 8, 128), lambda i: (0, 0, i, 0)),
            pl.BlockSpec((ROW_BLK, D_FEAT), lambda i: (i, 0)),
            pl.BlockSpec((2 * D_EDGE, D_FEAT), lambda i: (0, 0)),
            pl.BlockSpec((D_FEAT, D_FEAT), lambda i: (0, 0)),
            pl.BlockSpec((1, D_FEAT), lambda i: (0, 0)),
        ],
        out_specs=pl.BlockSpec((ROW_BLK, D_FEAT), lambda i: (i, 0)),
        out_shape=jax.ShapeDtypeStruct((N_NODES, D_FEAT), jnp.float32),
    )(agg, x, wio, wx, b2)
    return out


# speedup vs baseline: 1.0067x; 1.0067x over previous
import functools

import jax
import jax.numpy as jnp
from jax import lax
from jax.experimental import pallas as pl
from jax.experimental.pallas import tpu as pltpu
from jax.experimental.pallas import tpu_sc as plsc

N_NODES = 10000
N_EDGES = 320000
D_FEAT = 128
D_EDGE = 16

NC = 2
NS = 16

N_EB = N_EDGES // 128
CH = 125
N_CH = N_EB // CH
N_PAD = 10240
NRB = N_PAD // 128


def _sc_body(e4_hbm, idx_hbm, out_hbm,
             idx_sh, val_v, idx_v, acc, stg, vs0, vs1, is0, is1):
    c = lax.axis_index("c")
    s = lax.axis_index("s")
    fh = s // 8
    fm = lax.rem(s, 8)

    @pl.when(s == 0)
    def _():
        pltpu.sync_copy(idx_hbm, idx_sh)

    def _zero(i, carry):
        acc[pl.ds(i * 16, 16)] = jnp.zeros((16,), jnp.float32)
        return carry
    lax.fori_loop(0, N_PAD // 16, _zero, 0)
    plsc.subcore_barrier()

    vsems = (vs0, vs1)
    isems = (is0, is1)

    def _fetch(ch, b):
        pltpu.async_copy(
            e4_hbm.at[fh, pl.ds(ch * CH, CH), pl.ds(fm, 1), :],
            val_v.at[b], vsems[b])
        pltpu.async_copy(idx_sh.at[pl.ds(ch * CH, CH), pl.ds(1 - c, 1), :],
                         idx_v.at[b], isems[b])

    def _process(b):
        pltpu.make_async_copy(
            e4_hbm.at[fh, pl.ds(0, CH), pl.ds(fm, 1), :],
            val_v.at[b], vsems[b]).wait()
        pltpu.make_async_copy(idx_sh.at[pl.ds(0, CH), pl.ds(0, 1), :],
                              idx_v.at[b], isems[b]).wait()

        def _row(r, carry):
            ivs = [idx_v[b, r, 0, pl.ds(u * 16, 16)] for u in range(8)]
            vvs = [val_v[b, r, 0, pl.ds(u * 16, 16)] for u in range(8)]
            for u in range(8):
                plsc.addupdate_scatter(acc, [ivs[u]], vvs[u])
            return carry
        lax.fori_loop(0, CH, _row, 0)

    _fetch(0, 0)
    _fetch(1, 1)

    def _pair(t, carry):
        _process(0)

        @pl.when(2 * t + 2 < N_CH)
        def _():
            _fetch(2 * t + 2, 0)
        _process(1)

        @pl.when(2 * t + 3 < N_CH)
        def _():
            _fetch(2 * t + 3, 1)
        return carry
    lax.fori_loop(0, N_CH // 2, _pair, 0)

    def _rp(r, carry):
        for u in range(8):
            stg[r, pl.ds(u * 16, 16)] = acc[pl.ds(r * 128 + u * 16, 16)]
        return carry
    lax.fori_loop(0, NRB, _rp, 0)
    pltpu.sync_copy(stg, out_hbm.at[c, s])


_sc_agg = pl.kernel(
    _sc_body,
    out_type=jax.ShapeDtypeStruct((NC, NS, NRB, 128), jnp.float32),
    mesh=plsc.VectorSubcoreMesh(core_axis_name="c", subcore_axis_name="s"),
    compiler_params=pltpu.CompilerParams(use_tc_tiling_on_sc=False,
                                         needs_layout_passes=False),
    scratch_types=[
        pltpu.VMEM_SHARED((N_EB, 2, 128), jnp.int32),
        pltpu.VMEM((2, CH, 1, 128), jnp.float32),
        pltpu.VMEM((2, CH, 1, 128), jnp.int32),
        pltpu.VMEM((N_PAD,), jnp.float32),
        pltpu.VMEM((NRB, 128), jnp.float32),
        pltpu.SemaphoreType.DMA,
        pltpu.SemaphoreType.DMA,
        pltpu.SemaphoreType.DMA,
        pltpu.SemaphoreType.DMA,
    ],
)


ROW_BLK = 1024


def _mlp_body(agg_ref, x_ref, wio_ref, wx_ref, b_ref, o_ref):
    base = jnp.dot(x_ref[...], wx_ref[...],
                   preferred_element_type=jnp.float32) + b_ref[...]
    ac = agg_ref[...].reshape(2 * D_EDGE, 8, 128)
    for j in range(8):
        aj = ac[:, j, :]
        part = lax.dot_general(aj, wio_ref[...], (((0,), (0,)), ((), ())),
                               preferred_element_type=jnp.float32)
        o_ref[pl.ds(j * 128, 128), :] = base[j * 128:(j + 1) * 128, :] + part


@functools.partial(jax.jit, static_argnames=())
def kernel(x, edge_attr, W, b, edge_index):
    e4 = edge_attr.T.reshape(NC, 8, N_EB, 128).transpose(0, 2, 1, 3)
    idx3 = edge_index.astype(jnp.int32).reshape(2, N_EB, 128).transpose(1, 0, 2)
    agg = _sc_agg(e4, idx3)

    wio = W[:2 * D_EDGE]
    wx = W[2 * D_EDGE:]
    b2 = b.reshape(1, D_FEAT)

    grid = (N_PAD // ROW_BLK,)
    out = pl.pallas_call(
        _mlp_body,
        grid=grid,
        in_specs=[
            pl.BlockSpec((NC, NS, 8, 128), lambda i: (0, 0, i, 0)),
            pl.BlockSpec((ROW_BLK, D_FEAT), lambda i: (i, 0)),
            pl.BlockSpec((2 * D_EDGE, D_FEAT), lambda i: (0, 0)),
            pl.BlockSpec((D_FEAT, D_FEAT), lambda i: (0, 0)),
            pl.BlockSpec((1, D_FEAT), lambda i: (0, 0)),
        ],
        out_specs=pl.BlockSpec((ROW_BLK, D_FEAT), lambda i: (i, 0)),
        out_shape=jax.ShapeDtypeStruct((N_NODES, D_FEAT), jnp.float32),
    )(agg, x, wio, wx, b2)
    return out
